# Initial kernel scaffold; baseline (speedup 1.0000x reference)
import jax
import jax.numpy as jnp
from jax.experimental import pallas as pl


def kernel(x, edge_index, W_self, b_self, W_neigh, b_neigh, W_out, b_out):
    raise NotImplementedError("write your pallas kernel here")


# speedup vs baseline: 2.8090x; 2.8090x over previous
import jax
import jax.numpy as jnp
from jax import lax
from jax.experimental import pallas as pl
from jax.experimental.pallas import tpu as pltpu
from jax.experimental.pallas import tpu_sc as plsc

N = 10000
E = 160000
D_IN = 256
D_HID = 512
D_OUT = 256

DH = 128
NP = 10240
NTILES = 16
EDGES_PER_TILE = 10240
EP = NTILES * EDGES_PER_TILE
BATCH = 128
NBATCH = EDGES_PER_TILE // BATCH
ROWS_PER_TILE = NP // NTILES

DEG_EDGES_PER_TILE = EP // 32
DEG_NBATCH = DEG_EDGES_PER_TILE // BATCH


def _agg_body(xcat, src, dst, neigh_out, srcv, dstv, rows, zerov, acc, sem):
    c = lax.axis_index("c")
    s = lax.axis_index("s")
    coff = c * N

    zeros16 = jnp.zeros((16,), jnp.float32)

    def fill_zero(r, _):
        for j in range(DH // 16):
            zerov[r, pl.ds(j * 16, 16)] = zeros16
        return 0

    lax.fori_loop(0, 16, fill_zero, 0)

    rbase = s * ROWS_PER_TILE

    def zbody(j, _):
        pltpu.sync_copy(zerov, acc.at[pl.ds(rbase + j * 16, 16)])
        return 0

    lax.fori_loop(0, ROWS_PER_TILE // 16, zbody, 0)

    plsc.subcore_barrier()

    ebase = s * EDGES_PER_TILE

    def body(b, _):
        off = ebase + b * BATCH
        pltpu.sync_copy(src.at[pl.ds(off, BATCH)], srcv)
        pltpu.sync_copy(dst.at[pl.ds(off, BATCH)], dstv)
        for j in range(BATCH // 16):
            srcv[pl.ds(j * 16, 16)] = srcv[pl.ds(j * 16, 16)] + coff
        pltpu.async_copy(xcat.at[srcv], rows, sem).wait()
        pltpu.sync_copy(rows, acc.at[dstv], add=True)
        return 0

    lax.fori_loop(0, NBATCH, body, 0)

    plsc.subcore_barrier()

    pltpu.sync_copy(acc.at[pl.ds(rbase, ROWS_PER_TILE)],
                    neigh_out.at[c, pl.ds(rbase, ROWS_PER_TILE)])


_agg = pl.kernel(
    _agg_body,
    mesh=plsc.VectorSubcoreMesh(core_axis_name="c", subcore_axis_name="s"),
    out_type=[
        jax.ShapeDtypeStruct((2, NP, DH), jnp.float32),
    ],
    scratch_types=[
        pltpu.VMEM((BATCH,), jnp.int32),
        pltpu.VMEM((BATCH,), jnp.int32),
        pltpu.VMEM((BATCH, DH), jnp.float32),
        pltpu.VMEM((16, DH), jnp.float32),
        pltpu.VMEM_SHARED((NP, DH), jnp.float32),
        pltpu.SemaphoreType.DMA,
    ],
)


def _deg_body(dst, deg_out, dstv, onesv, zdv, dacc):
    c = lax.axis_index("c")
    s = lax.axis_index("s")

    zeros16 = jnp.zeros((16,), jnp.float32)
    ones16 = jnp.ones((16,), jnp.float32)

    def fill(r, _):
        for j in range(DH // 16):
            zdv[r, pl.ds(j * 16, 16)] = zeros16
        return 0

    lax.fori_loop(0, 16, fill, 0)

    def fill_ones(r, _):
        for j in range(DH // 16):
            onesv[r, pl.ds(j * 16, 16)] = ones16
        return 0

    lax.fori_loop(0, BATCH, fill_ones, 0)

    rbase = s * ROWS_PER_TILE

    def zbody(j, _):
        pltpu.sync_copy(zdv, dacc.at[pl.ds(rbase + j * 16, 16)])
        return 0

    lax.fori_loop(0, ROWS_PER_TILE // 16, zbody, 0)

    plsc.subcore_barrier()

    ebase = (c * NTILES + s) * DEG_EDGES_PER_TILE

    def body(b, _):
        off = ebase + b * BATCH
        pltpu.sync_copy(dst.at[pl.ds(off, BATCH)], dstv)
        pltpu.sync_copy(onesv, dacc.at[dstv], add=True)
        return 0

    lax.fori_loop(0, DEG_NBATCH, body, 0)

    plsc.subcore_barrier()

    pltpu.sync_copy(dacc.at[pl.ds(rbase, ROWS_PER_TILE)],
                    deg_out.at[c, pl.ds(rbase, ROWS_PER_TILE)])


_deg = pl.kernel(
    _deg_body,
    mesh=plsc.VectorSubcoreMesh(core_axis_name="c", subcore_axis_name="s"),
    out_type=[
        jax.ShapeDtypeStruct((2, NP, DH), jnp.float32),
    ],
    scratch_types=[
        pltpu.VMEM((BATCH,), jnp.int32),
        pltpu.VMEM((BATCH, DH), jnp.float32),
        pltpu.VMEM((16, DH), jnp.float32),
        pltpu.VMEM_SHARED((NP, DH), jnp.float32),
    ],
)


BLK = 1000


def _mlp_body(xb, n2b, db, ws, bs, wn, bn, wo, bo, ob):
    xv = xb[...]
    h_self = jnp.dot(xv, ws[...], preferred_element_type=jnp.float32) + bs[...]
    n2 = n2b[...]
    nb = jnp.concatenate([n2[0], n2[1]], axis=-1)
    d2 = db[...]
    deg = jnp.maximum(d2[0][:, 0:1] + d2[1][:, 0:1], 1.0)
    h_neigh = jnp.dot(nb / deg, wn[...],
                      preferred_element_type=jnp.float32) + bn[...]
    h = jnp.maximum(h_self + h_neigh, 0.0)
    ob[...] = jnp.dot(h, wo[...], preferred_element_type=jnp.float32) + bo[...]


def _mlp(x, neigh2, deg2, ws_t, bs, wn_t, bn, wo_t, bo):
    grid = (N // BLK,)
    return pl.pallas_call(
        _mlp_body,
        grid=grid,
        in_specs=[
            pl.BlockSpec((BLK, D_IN), lambda i: (i, 0)),
            pl.BlockSpec((2, BLK, DH), lambda i: (0, i, 0)),
            pl.BlockSpec((2, BLK, DH), lambda i: (0, i, 0)),
            pl.BlockSpec((D_IN, D_HID), lambda i: (0, 0)),
            pl.BlockSpec((1, D_HID), lambda i: (0, 0)),
            pl.BlockSpec((D_IN, D_HID), lambda i: (0, 0)),
            pl.BlockSpec((1, D_HID), lambda i: (0, 0)),
            pl.BlockSpec((D_HID, D_OUT), lambda i: (0, 0)),
            pl.BlockSpec((1, D_OUT), lambda i: (0, 0)),
        ],
        out_specs=pl.BlockSpec((BLK, D_OUT), lambda i: (i, 0)),
        out_shape=jax.ShapeDtypeStruct((N, D_OUT), jnp.float32),
    )(x, neigh2, deg2, ws_t, bs, wn_t, bn, wo_t, bo)


def kernel(x, edge_index, W_self, b_self, W_neigh, b_neigh, W_out, b_out):
    xcat = jnp.concatenate([x[:, :DH], x[:, DH:]], axis=0)
    src = edge_index[0]
    dst = edge_index[1]
    pad = EP - E
    srcp = jnp.concatenate([src, jnp.zeros((pad,), jnp.int32)])
    dstp = jnp.concatenate([dst, jnp.full((pad,), N, jnp.int32)])
    (neigh2,) = _agg(xcat, srcp, dstp)
    (deg2,) = _deg(dstp)
    return _mlp(x, neigh2, deg2, W_self.T, b_self[None, :],
                W_neigh.T, b_neigh[None, :], W_out.T, b_out[None, :])


# speedup vs baseline: 3.9821x; 1.4176x over previous
import jax
import jax.numpy as jnp
from jax import lax
from jax.experimental import pallas as pl
from jax.experimental.pallas import tpu as pltpu
from jax.experimental.pallas import tpu_sc as plsc

N = 10000
E = 160000
D_IN = 256
D_HID = 512
D_OUT = 256

DH = 128
NP = 10240
NTILES = 16
EDGES_PER_TILE = 10240
EP = NTILES * EDGES_PER_TILE
BATCH = 128
NBATCH = EDGES_PER_TILE // BATCH
ROWS_PER_TILE = NP // NTILES

DEG_EDGES_PER_TILE = EP // 32
DEG_NBATCH = DEG_EDGES_PER_TILE // BATCH


def _agg_body(xcat, src2, dst2, neigh_out,
              srcv, dstv, rows0, rows1, acc, semg0, semg1):
    c = lax.axis_index("c")
    s = lax.axis_index("s")
    coff = c * N

    zeros16 = jnp.zeros((16,), jnp.float32)

    def fill_zero(r, _):
        for j in range(DH // 16):
            rows0[r, pl.ds(j * 16, 16)] = zeros16
        return 0

    lax.fori_loop(0, BATCH, fill_zero, 0)

    rbase = s * ROWS_PER_TILE

    def zbody(j, _):
        pltpu.sync_copy(rows0, acc.at[pl.ds(rbase + j * BATCH, BATCH)])
        return 0

    lax.fori_loop(0, ROWS_PER_TILE // BATCH, zbody, 0)

    plsc.subcore_barrier()

    HB = NBATCH // 2
    HE = EDGES_PER_TILE // 2

    def gstart(b, buf, sem):
        pltpu.async_copy(xcat.at[srcv.at[pl.ds(b * BATCH, BATCH)]],
                         buf, sem)

    def gwait(b, buf, sem):
        pltpu.make_async_copy(xcat.at[srcv.at[pl.ds(b * BATCH, BATCH)]],
                              buf, sem).wait()

    def scat(b, buf):
        pltpu.sync_copy(buf, acc.at[dstv.at[b]], add=True)

    for h in range(2):
        pltpu.sync_copy(src2.at[s, pl.ds(h * HE, HE)], srcv)
        pltpu.sync_copy(dst2.at[s, pl.ds(h * HB, HB)], dstv)

        def adj(i, _):
            srcv[pl.ds(i * 16, 16)] = srcv[pl.ds(i * 16, 16)] + coff
            return 0

        lax.fori_loop(0, HE // 16, adj, 0)

        gstart(0, rows0, semg0)

        def body2(i, _):
            b0 = 2 * i
            b1 = b0 + 1
            gstart(b1, rows1, semg1)
            gwait(b0, rows0, semg0)
            scat(b0, rows0)

            @pl.when(b1 < HB - 1)
            def _():
                gstart(b0 + 2, rows0, semg0)

            gwait(b1, rows1, semg1)
            scat(b1, rows1)
            return 0

        lax.fori_loop(0, HB // 2, body2, 0)

    plsc.subcore_barrier()

    pltpu.sync_copy(acc.at[pl.ds(rbase, ROWS_PER_TILE)],
                    neigh_out.at[c, pl.ds(rbase, ROWS_PER_TILE)])


_agg = pl.kernel(
    _agg_body,
    mesh=plsc.VectorSubcoreMesh(core_axis_name="c", subcore_axis_name="s"),
    out_type=[
        jax.ShapeDtypeStruct((2, NP, DH), jnp.float32),
    ],
    scratch_types=[
        pltpu.VMEM((EDGES_PER_TILE // 2,), jnp.int32),
        pltpu.VMEM((NBATCH // 2, BATCH), jnp.int32),
        pltpu.VMEM((BATCH, DH), jnp.float32),
        pltpu.VMEM((BATCH, DH), jnp.float32),
        pltpu.VMEM_SHARED((NP, DH), jnp.float32),
        pltpu.SemaphoreType.DMA,
        pltpu.SemaphoreType.DMA,
    ],
)


def _deg_body(dst3, deg_out, dstv, onesv, zdv, dacc):
    c = lax.axis_index("c")
    s = lax.axis_index("s")

    zeros16 = jnp.zeros((16,), jnp.float32)
    ones16 = jnp.ones((16,), jnp.float32)

    def fill(r, _):
        for j in range(DH // 16):
            zdv[r, pl.ds(j * 16, 16)] = zeros16
        return 0

    lax.fori_loop(0, 16, fill, 0)

    def fill_ones(r, _):
        for j in range(DH // 16):
            onesv[r, pl.ds(j * 16, 16)] = ones16
        return 0

    lax.fori_loop(0, BATCH, fill_ones, 0)

    pltpu.sync_copy(dst3.at[c * NTILES + s], dstv)

    rbase = s * ROWS_PER_TILE

    def zbody(j, _):
        pltpu.sync_copy(zdv, dacc.at[pl.ds(rbase + j * 16, 16)])
        return 0

    lax.fori_loop(0, ROWS_PER_TILE // 16, zbody, 0)

    plsc.subcore_barrier()

    def body(b, _):
        pltpu.sync_copy(onesv, dacc.at[dstv.at[b]], add=True)
        return 0

    lax.fori_loop(0, DEG_NBATCH, body, 0)

    plsc.subcore_barrier()

    pltpu.sync_copy(dacc.at[pl.ds(rbase, ROWS_PER_TILE)],
                    deg_out.at[c, pl.ds(rbase, ROWS_PER_TILE)])


_deg = pl.kernel(
    _deg_body,
    mesh=plsc.VectorSubcoreMesh(core_axis_name="c", subcore_axis_name="s"),
    out_type=[
        jax.ShapeDtypeStruct((2, NP, DH), jnp.float32),
    ],
    scratch_types=[
        pltpu.VMEM((DEG_NBATCH, BATCH), jnp.int32),
        pltpu.VMEM((BATCH, DH), jnp.float32),
        pltpu.VMEM((16, DH), jnp.float32),
        pltpu.VMEM_SHARED((NP, DH), jnp.float32),
    ],
)


BLK = 1000


def _mlp_body(xb, n2b, db, ws, bs, wn, bn, wo, bo, ob):
    xv = xb[...]
    h_self = jnp.dot(xv, ws[...], preferred_element_type=jnp.float32) + bs[...]
    n2 = n2b[...]
    nb = jnp.concatenate([n2[0], n2[1]], axis=-1)
    d2 = db[...]
    deg = jnp.maximum(d2[0][:, 0:1] + d2[1][:, 0:1], 1.0)
    h_neigh = jnp.dot(nb / deg, wn[...],
                      preferred_element_type=jnp.float32) + bn[...]
    h = jnp.maximum(h_self + h_neigh, 0.0)
    ob[...] = jnp.dot(h, wo[...], preferred_element_type=jnp.float32) + bo[...]


def _mlp(x, neigh2, deg2, ws_t, bs, wn_t, bn, wo_t, bo):
    grid = (N // BLK,)
    return pl.pallas_call(
        _mlp_body,
        grid=grid,
        in_specs=[
            pl.BlockSpec((BLK, D_IN), lambda i: (i, 0)),
            pl.BlockSpec((2, BLK, DH), lambda i: (0, i, 0)),
            pl.BlockSpec((2, BLK, DH), lambda i: (0, i, 0)),
            pl.BlockSpec((D_IN, D_HID), lambda i: (0, 0)),
            pl.BlockSpec((1, D_HID), lambda i: (0, 0)),
            pl.BlockSpec((D_IN, D_HID), lambda i: (0, 0)),
            pl.BlockSpec((1, D_HID), lambda i: (0, 0)),
            pl.BlockSpec((D_HID, D_OUT), lambda i: (0, 0)),
            pl.BlockSpec((1, D_OUT), lambda i: (0, 0)),
        ],
        out_specs=pl.BlockSpec((BLK, D_OUT), lambda i: (i, 0)),
        out_shape=jax.ShapeDtypeStruct((N, D_OUT), jnp.float32),
    )(x, neigh2, deg2, ws_t, bs, wn_t, bn, wo_t, bo)


def kernel(x, edge_index, W_self, b_self, W_neigh, b_neigh, W_out, b_out):
    xcat = jnp.concatenate([x[:, :DH], x[:, DH:]], axis=0)
    src = edge_index[0]
    dst = edge_index[1]
    pad = EP - E
    srcp = jnp.concatenate([src, jnp.zeros((pad,), jnp.int32)])
    dstp = jnp.concatenate([dst, jnp.full((pad,), N, jnp.int32)])
    src2 = srcp.reshape(NTILES, EDGES_PER_TILE)
    dst2 = dstp.reshape(NTILES, NBATCH, BATCH)
    dst3 = dstp.reshape(2 * NTILES, DEG_NBATCH, BATCH)
    (neigh2,) = _agg(xcat, src2, dst2)
    (deg2,) = _deg(dst3)
    return _mlp(x, neigh2, deg2, W_self.T, b_self[None, :],
                W_neigh.T, b_neigh[None, :], W_out.T, b_out[None, :])
